# baseline (speedup 1.0000x reference)
import functools

import jax
import jax.numpy as jnp
from jax import lax
from jax.experimental import pallas as pl
from jax.experimental.pallas import tpu as pltpu
from jax.experimental.pallas import tpu_sc as plsc

_TC_BLOCKS = (256, 256, 512, 1024, 2048)
_SC_CHUNK = 64


def _tc_part(W_pos, batch, seq_len, d_model):
    offs = [0]
    for s in _TC_BLOCKS:
        offs.append(offs[-1] + s)
    nblk = len(_TC_BLOCKS)

    def _dma_kernel(w_hbm, o_hbm, buf, in_sems, out_sems):
        def in_copy(i):
            return pltpu.make_async_copy(
                w_hbm.at[pl.ds(offs[i], _TC_BLOCKS[i])],
                buf.at[pl.ds(offs[i], _TC_BLOCKS[i])],
                in_sems.at[i],
            )

        def out_copy(i, b):
            return pltpu.make_async_copy(
                buf.at[pl.ds(offs[i], _TC_BLOCKS[i])],
                o_hbm.at[b, pl.ds(offs[i], _TC_BLOCKS[i])],
                out_sems.at[i, b],
            )

        for i in range(nblk):
            in_copy(i).start()
        for i in range(nblk):
            in_copy(i).wait()
            for b in range(batch):
                out_copy(i, b).start()
        for i in range(nblk):
            for b in range(batch):
                out_copy(i, b).wait()

    return pl.pallas_call(
        _dma_kernel,
        in_specs=[pl.BlockSpec(memory_space=pl.ANY)],
        out_specs=pl.BlockSpec(memory_space=pl.ANY),
        out_shape=jax.ShapeDtypeStruct((batch, seq_len, d_model), W_pos.dtype),
        scratch_shapes=[
            pltpu.VMEM((seq_len, d_model), W_pos.dtype),
            pltpu.SemaphoreType.DMA((nblk,)),
            pltpu.SemaphoreType.DMA((nblk, batch)),
        ],
    )(W_pos)


def _sc_part(W_pos, batch, seq_len, d_model):
    info = plsc.get_sparse_core_info()
    nc, ns = info.num_cores, info.num_subcores
    nw = nc * ns
    rows_per_w = (batch * seq_len) // nw
    seq_per_b = seq_len // rows_per_w
    nchunk = rows_per_w // _SC_CHUNK

    mesh = plsc.VectorSubcoreMesh(core_axis_name="c", subcore_axis_name="s")

    @functools.partial(
        pl.kernel,
        mesh=mesh,
        out_type=jax.ShapeDtypeStruct((batch, seq_len, d_model), jnp.float32),
        scratch_types=[
            pltpu.VMEM((2, _SC_CHUNK, d_model), jnp.float32),
            pltpu.SemaphoreType.DMA((2,)),
            pltpu.SemaphoreType.DMA((2,)),
        ],
    )
    def _sc_copy(w_hbm, out_hbm, buf, in_sems, out_sems):
        wid = lax.axis_index("s") * nc + lax.axis_index("c")
        b = wid // seq_per_b
        s0 = (wid % seq_per_b) * rows_per_w

        def in_copy(j):
            return pltpu.make_async_copy(
                w_hbm.at[pl.ds(s0 + j * _SC_CHUNK, _SC_CHUNK)],
                buf.at[j % 2],
                in_sems.at[j % 2],
            )

        def out_copy(j):
            return pltpu.make_async_copy(
                buf.at[j % 2],
                out_hbm.at[b, pl.ds(s0 + j * _SC_CHUNK, _SC_CHUNK)],
                out_sems.at[j % 2],
            )

        in_copy(0).start()
        for j in range(nchunk):
            in_copy(j).wait()
            out_copy(j).start()
            if j + 1 < nchunk:
                if j >= 1:
                    out_copy(j - 1).wait()
                in_copy(j + 1).start()
        if nchunk >= 2:
            out_copy(nchunk - 2).wait()
        out_copy(nchunk - 1).wait()

    return _sc_copy(W_pos)


def kernel(tokens, W_pos):
    batch, seq_len = tokens.shape
    d_model = W_pos.shape[1]
    b_sc = 1
    b_tc = batch - b_sc
    out_tc = _tc_part(W_pos, b_tc, seq_len, d_model)
    out_sc = _sc_part(W_pos, b_sc, seq_len, d_model)
    return jnp.concatenate([out_tc, out_sc], axis=0)


# speedup vs baseline: 3.5492x; 3.5492x over previous
import jax
import jax.numpy as jnp
from jax.experimental import pallas as pl
from jax.experimental.pallas import tpu as pltpu

_BLOCKS = (128, 128, 256, 512, 1024, 2048)


def kernel(tokens, W_pos):
    batch, seq_len = tokens.shape
    d_model = W_pos.shape[1]
    assert sum(_BLOCKS) == seq_len
    offs = [0]
    for s in _BLOCKS:
        offs.append(offs[-1] + s)
    nblk = len(_BLOCKS)

    def _dma_kernel(w_hbm, o_hbm, buf, in_sems, out_sems):
        def in_copy(i):
            return pltpu.make_async_copy(
                w_hbm.at[pl.ds(offs[i], _BLOCKS[i])],
                buf.at[pl.ds(offs[i], _BLOCKS[i])],
                in_sems.at[i],
            )

        def out_copy(i, b):
            return pltpu.make_async_copy(
                buf.at[pl.ds(offs[i], _BLOCKS[i])],
                o_hbm.at[b, pl.ds(offs[i], _BLOCKS[i])],
                out_sems.at[i, b],
            )

        for i in range(nblk):
            in_copy(i).start()
        for i in range(nblk):
            in_copy(i).wait()
            for b in range(batch):
                out_copy(i, b).start()
        for i in range(nblk):
            for b in range(batch):
                out_copy(i, b).wait()

    out = pl.pallas_call(
        _dma_kernel,
        in_specs=[pl.BlockSpec(memory_space=pl.ANY)],
        out_specs=pl.BlockSpec(memory_space=pl.ANY),
        out_shape=jax.ShapeDtypeStruct((batch, seq_len, d_model), W_pos.dtype),
        scratch_shapes=[
            pltpu.VMEM((seq_len, d_model), W_pos.dtype),
            pltpu.SemaphoreType.DMA((nblk,)),
            pltpu.SemaphoreType.DMA((nblk, batch)),
        ],
    )(W_pos)
    return out


# speedup vs baseline: 3.6052x; 1.0158x over previous
import jax
import jax.numpy as jnp
from jax.experimental import pallas as pl
from jax.experimental.pallas import tpu as pltpu

_BLOCKS = (1024, 1024, 2048)


def kernel(tokens, W_pos):
    batch, seq_len = tokens.shape
    d_model = W_pos.shape[1]
    assert sum(_BLOCKS) == seq_len
    offs = [0]
    for s in _BLOCKS:
        offs.append(offs[-1] + s)
    nblk = len(_BLOCKS)

    def _dma_kernel(w_hbm, o_hbm, buf, in_sems, out_sems):
        def in_copy(i):
            return pltpu.make_async_copy(
                w_hbm.at[pl.ds(offs[i], _BLOCKS[i])],
                buf.at[pl.ds(offs[i], _BLOCKS[i])],
                in_sems.at[i],
            )

        def out_copy(i, b):
            return pltpu.make_async_copy(
                buf.at[pl.ds(offs[i], _BLOCKS[i])],
                o_hbm.at[b, pl.ds(offs[i], _BLOCKS[i])],
                out_sems.at[i, b],
            )

        for i in range(nblk):
            in_copy(i).start()
        for i in range(nblk):
            in_copy(i).wait()
            for b in range(batch):
                out_copy(i, b).start()
        for i in range(nblk):
            for b in range(batch):
                out_copy(i, b).wait()

    out = pl.pallas_call(
        _dma_kernel,
        in_specs=[pl.BlockSpec(memory_space=pl.ANY)],
        out_specs=pl.BlockSpec(memory_space=pl.ANY),
        out_shape=jax.ShapeDtypeStruct((batch, seq_len, d_model), W_pos.dtype),
        scratch_shapes=[
            pltpu.VMEM((seq_len, d_model), W_pos.dtype),
            pltpu.SemaphoreType.DMA((nblk,)),
            pltpu.SemaphoreType.DMA((nblk, batch)),
        ],
    )(W_pos)
    return out
